# baseline (speedup 1.0000x reference)
import jax
import jax.numpy as jnp
from jax import lax
from jax.experimental import pallas as pl
from jax.experimental.pallas import tpu as pltpu
from jax.experimental.pallas import tpu_sc as plsc

_N = 10000
_E = 320000
_D = 128

_NSC = 2
_NTEC = 16
_CHUNK = 128
_GRP = 32
_CPS = 160
_CREC = _CPS
_RPT = 640
_ZR = 80

_LANES = 16
_VPR = _D // _LANES


def _sc_spmm_body(x_hbm, sdw_hbm, out_hbm,
                  rows_v, sdw_v, acc_sh, gsem, ssem):
    c = lax.axis_index("c")
    s = lax.axis_index("s")

    def _zero_row(r, _):
        for j in range(_VPR):
            rows_v[0, r, pl.ds(j * _LANES, _LANES)] = jnp.zeros(
                (_LANES,), jnp.float32)
        return 0

    lax.fori_loop(0, _ZR, _zero_row, 0)

    @pl.when(s < _NTEC - 1)
    def _zero_full():
        for r in range(_RPT // _ZR):
            pltpu.sync_copy(rows_v.at[0, pl.ds(0, _ZR)],
                            acc_sh.at[pl.ds(s * _RPT + r * _ZR, _ZR)])

    @pl.when(s == _NTEC - 1)
    def _zero_tail():
        for r in range((_N - (_NTEC - 1) * _RPT) // _ZR):
            pltpu.sync_copy(rows_v.at[0, pl.ds(0, _ZR)],
                            acc_sh.at[pl.ds(s * _RPT + r * _ZR, _ZR)])

    plsc.subcore_barrier()

    def _blk(gi, _):
        pltpu.sync_copy(sdw_hbm.at[c, s, pl.ds(gi * _GRP, _GRP)], sdw_v)

        def _slot(u, _):
            pltpu.async_copy(x_hbm.at[sdw_v.at[u, 0]], rows_v.at[0],
                             gsem).wait()

            def _group(g, _):
                wb = sdw_v[u, 2, pl.ds(g * _LANES, _LANES)]
                for k in range(_LANES):
                    e = g * _LANES + k
                    wv = jnp.full((_LANES,),
                                  lax.bitcast_convert_type(wb[k],
                                                           jnp.float32))
                    for j in range(_VPR):
                        sl = pl.ds(j * _LANES, _LANES)
                        rows_v[0, e, sl] = rows_v[0, e, sl] * wv
                return 0

            lax.fori_loop(0, _CHUNK // _LANES, _group, 0)
            pltpu.sync_copy(rows_v.at[0], acc_sh.at[sdw_v.at[u, 1]],
                            add=True)
            return 0

        lax.fori_loop(0, _GRP, _slot, 0)
        return 0

    lax.fori_loop(0, _CPS // _GRP, _blk, 0)
    plsc.subcore_barrier()

    @pl.when(s < _NTEC - 1)
    def _write_full():
        pltpu.sync_copy(acc_sh.at[pl.ds(s * _RPT, _RPT)],
                        out_hbm.at[c, pl.ds(s * _RPT, _RPT)])

    @pl.when(s == _NTEC - 1)
    def _write_tail():
        tail = _N - (_NTEC - 1) * _RPT
        pltpu.sync_copy(acc_sh.at[pl.ds((_NTEC - 1) * _RPT, tail)],
                        out_hbm.at[c, pl.ds((_NTEC - 1) * _RPT, tail)])


def _sc_spmm(x, sdw):
    return pl.kernel(
        _sc_spmm_body,
        out_type=jax.ShapeDtypeStruct((_NSC, _N, _D), jnp.float32),
        mesh=plsc.VectorSubcoreMesh(core_axis_name="c",
                                    subcore_axis_name="s"),
        scratch_types=[
            pltpu.VMEM((1, _CHUNK, _D), jnp.float32),
            pltpu.VMEM((_GRP, 3, _CHUNK), jnp.int32),
            pltpu.VMEM_SHARED((_N, _D), jnp.float32),
            pltpu.SemaphoreType.DMA,
            pltpu.SemaphoreType.DMA,
        ],
    )(x, sdw)


_BLK = 1000
_SCALE = float(_D) ** 0.5


def _tc_readout_body(h_ref, a_ref, b_ref, w1_ref, b1_ref, w2_ref, b2_ref,
                     cw0_ref, cw1_ref, cb_ref, o_ref):
    hb = h_ref[...]
    a = a_ref[...]
    b = b_ref[...]
    t1 = jnp.maximum(
        jnp.dot(hb * a, w1_ref[...], preferred_element_type=jnp.float32)
        + b1_ref[...], 0.0)
    t2 = jnp.maximum(
        jnp.dot(hb * b, w2_ref[...], preferred_element_type=jnp.float32)
        + b2_ref[...], 0.0)
    s1 = jnp.sum(t1, axis=1, keepdims=True) * (1.0 / _SCALE)
    s2 = jnp.sum(t2, axis=1, keepdims=True) * (1.0 / _SCALE)
    m = jnp.maximum(s1, s2)
    e1 = jnp.exp(s1 - m)
    e2 = jnp.exp(s2 - m)
    r1 = e1 / (e1 + e2)
    nb = a * r1 + b * (1.0 - r1)
    o_ref[...] = hb * cw0_ref[...] + nb * cw1_ref[...] + cb_ref[...]


def _tc_readout(h, a, b, w1, b1, w2, b2, cw0, cw1, cb):
    full = pl.BlockSpec((_D, _D), lambda i: (0, 0))
    row = pl.BlockSpec((1, _D), lambda i: (0, 0))
    blk = pl.BlockSpec((_BLK, _D), lambda i: (i, 0))
    return pl.pallas_call(
        _tc_readout_body,
        grid=(_N // _BLK,),
        in_specs=[blk, blk, blk, full, row, full, row, row, row, row],
        out_specs=blk,
        out_shape=jax.ShapeDtypeStruct((_N, _D), jnp.float32),
    )(h, a, b, w1, b1, w2, b2, cw0, cw1, cb)


def kernel(x, edge_index_in, edge_weight_in, edge_index_out, edge_weight_out,
           W1_w, W1_b, W2_w, W2_b, conv_w, conv_b):
    pad = _CPS * _CHUNK * _NTEC - _E
    src = jnp.pad(jnp.stack([edge_index_in[1], edge_index_out[1]]),
                  ((0, 0), (0, pad))).reshape(_NSC, _NTEC, _CPS, 1, _CHUNK)
    dst = jnp.pad(jnp.stack([edge_index_in[0], edge_index_out[0]]),
                  ((0, 0), (0, pad))).reshape(_NSC, _NTEC, _CPS, 1, _CHUNK)
    wb = lax.bitcast_convert_type(
        jnp.pad(jnp.stack([edge_weight_in, edge_weight_out]),
                ((0, 0), (0, pad))),
        jnp.int32).reshape(_NSC, _NTEC, _CPS, 1, _CHUNK)
    sdw = jnp.pad(jnp.concatenate([src, dst, wb], axis=3),
                  ((0, 0), (0, 0), (0, _CREC - _CPS), (0, 0), (0, 0)))

    nbrs = _sc_spmm(x, sdw)

    out = _tc_readout(
        x, nbrs[0], nbrs[1],
        W1_w, W1_b.reshape(1, _D), W2_w, W2_b.reshape(1, _D),
        jnp.full((1, _D), conv_w[0]),
        jnp.full((1, _D), conv_w[1]),
        jnp.full((1, _D), conv_b))
    return out


# speedup vs baseline: 1.5070x; 1.5070x over previous
import jax
import jax.numpy as jnp
from jax import lax
from jax.experimental import pallas as pl
from jax.experimental.pallas import tpu as pltpu
from jax.experimental.pallas import tpu_sc as plsc

_N = 10000
_E = 320000
_D = 128

_NSC = 2
_NTEC = 16
_CHUNK = 128
_CPS = 157
_RPT = 640
_ZR = 80

_LANES = 16
_VPR = _D // _LANES


def _sc_spmm_body(x_hbm, sd_hbm, w_hbm, out_hbm,
                  rows_v, sd_v, w_v, acc_sh, gsem):
    c = lax.axis_index("c")
    s = lax.axis_index("s")

    def _zero_row(r, _):
        for j in range(_VPR):
            rows_v[r, pl.ds(j * _LANES, _LANES)] = jnp.zeros(
                (_LANES,), jnp.float32)
        return 0

    lax.fori_loop(0, _ZR, _zero_row, 0)

    @pl.when(s < _NTEC - 1)
    def _zero_full():
        for r in range(_RPT // _ZR):
            pltpu.sync_copy(rows_v.at[pl.ds(0, _ZR)],
                            acc_sh.at[pl.ds(s * _RPT + r * _ZR, _ZR)])

    @pl.when(s == _NTEC - 1)
    def _zero_tail():
        for r in range((_N - (_NTEC - 1) * _RPT) // _ZR):
            pltpu.sync_copy(rows_v.at[pl.ds(0, _ZR)],
                            acc_sh.at[pl.ds(s * _RPT + r * _ZR, _ZR)])

    plsc.subcore_barrier()

    pltpu.sync_copy(w_hbm.at[c, s], w_v)

    def _slot(t, _):
        pltpu.sync_copy(sd_hbm.at[c, s, t], sd_v)
        pltpu.async_copy(x_hbm.at[sd_v.at[0]], rows_v, gsem).wait()

        def _group(g, _):
            wvec = w_v[pl.ds(t * _CHUNK + g * _LANES, _LANES)]
            for k in range(_LANES):
                e = g * _LANES + k
                wv = jnp.full((_LANES,), wvec[k])
                for j in range(_VPR):
                    sl = pl.ds(j * _LANES, _LANES)
                    rows_v[e, sl] = rows_v[e, sl] * wv
            return 0

        lax.fori_loop(0, _CHUNK // _LANES, _group, 0)
        pltpu.sync_copy(rows_v, acc_sh.at[sd_v.at[1]], add=True)
        return 0

    lax.fori_loop(0, _CPS, _slot, 0)
    plsc.subcore_barrier()

    @pl.when(s < _NTEC - 1)
    def _write_full():
        pltpu.sync_copy(acc_sh.at[pl.ds(s * _RPT, _RPT)],
                        out_hbm.at[c, pl.ds(s * _RPT, _RPT)])

    @pl.when(s == _NTEC - 1)
    def _write_tail():
        tail = _N - (_NTEC - 1) * _RPT
        pltpu.sync_copy(acc_sh.at[pl.ds((_NTEC - 1) * _RPT, tail)],
                        out_hbm.at[c, pl.ds((_NTEC - 1) * _RPT, tail)])


def _sc_spmm(x, sd, w):
    return pl.kernel(
        _sc_spmm_body,
        out_type=jax.ShapeDtypeStruct((_NSC, _N, _D), jnp.float32),
        mesh=plsc.VectorSubcoreMesh(core_axis_name="c",
                                    subcore_axis_name="s"),
        scratch_types=[
            pltpu.VMEM((_CHUNK, _D), jnp.float32),
            pltpu.VMEM((2, _CHUNK), jnp.int32),
            pltpu.VMEM((_CPS * _CHUNK,), jnp.float32),
            pltpu.VMEM_SHARED((_N, _D), jnp.float32),
            pltpu.SemaphoreType.DMA,
        ],
    )(x, sd, w)


_BLK = 1000
_SCALE = float(_D) ** 0.5


def _tc_readout_body(h_ref, a_ref, b_ref, w1_ref, b1_ref, w2_ref, b2_ref,
                     cw0_ref, cw1_ref, cb_ref, o_ref):
    hb = h_ref[...]
    a = a_ref[...]
    b = b_ref[...]
    t1 = jnp.maximum(
        jnp.dot(hb * a, w1_ref[...], preferred_element_type=jnp.float32)
        + b1_ref[...], 0.0)
    t2 = jnp.maximum(
        jnp.dot(hb * b, w2_ref[...], preferred_element_type=jnp.float32)
        + b2_ref[...], 0.0)
    s1 = jnp.sum(t1, axis=1, keepdims=True) * (1.0 / _SCALE)
    s2 = jnp.sum(t2, axis=1, keepdims=True) * (1.0 / _SCALE)
    m = jnp.maximum(s1, s2)
    e1 = jnp.exp(s1 - m)
    e2 = jnp.exp(s2 - m)
    r1 = e1 / (e1 + e2)
    nb = a * r1 + b * (1.0 - r1)
    o_ref[...] = hb * cw0_ref[...] + nb * cw1_ref[...] + cb_ref[...]


def _tc_readout(h, a, b, w1, b1, w2, b2, cw0, cw1, cb):
    full = pl.BlockSpec((_D, _D), lambda i: (0, 0))
    row = pl.BlockSpec((1, _D), lambda i: (0, 0))
    blk = pl.BlockSpec((_BLK, _D), lambda i: (i, 0))
    return pl.pallas_call(
        _tc_readout_body,
        grid=(_N // _BLK,),
        in_specs=[blk, blk, blk, full, row, full, row, row, row, row],
        out_specs=blk,
        out_shape=jax.ShapeDtypeStruct((_N, _D), jnp.float32),
    )(h, a, b, w1, b1, w2, b2, cw0, cw1, cb)


def kernel(x, edge_index_in, edge_weight_in, edge_index_out, edge_weight_out,
           W1_w, W1_b, W2_w, W2_b, conv_w, conv_b):
    pad = _CPS * _CHUNK * _NTEC - _E
    src = jnp.pad(jnp.stack([edge_index_in[1], edge_index_out[1]]),
                  ((0, 0), (0, pad))).reshape(_NSC, _NTEC, _CPS, 1, _CHUNK)
    dst = jnp.pad(jnp.stack([edge_index_in[0], edge_index_out[0]]),
                  ((0, 0), (0, pad))).reshape(_NSC, _NTEC, _CPS, 1, _CHUNK)
    sd = jnp.concatenate([src, dst], axis=3)
    w = jnp.pad(jnp.stack([edge_weight_in, edge_weight_out]),
                ((0, 0), (0, pad))).reshape(_NSC, _NTEC, _CPS * _CHUNK)

    nbrs = _sc_spmm(x, sd, w)

    out = _tc_readout(
        x, nbrs[0], nbrs[1],
        W1_w, W1_b.reshape(1, _D), W2_w, W2_b.reshape(1, _D),
        jnp.full((1, _D), conv_w[0]),
        jnp.full((1, _D), conv_w[1]),
        jnp.full((1, _D), conv_b))
    return out


# speedup vs baseline: 1.5090x; 1.0013x over previous
import jax
import jax.numpy as jnp
from jax import lax
from jax.experimental import pallas as pl
from jax.experimental.pallas import tpu as pltpu
from jax.experimental.pallas import tpu_sc as plsc

_N = 10000
_E = 320000
_D = 128

_NSC = 2
_NTEC = 16
_CHUNK = 128
_CPS = 157
_RPT = 640
_ZR = 80

_LANES = 16
_VPR = _D // _LANES


def _sc_spmm_body(x_hbm, sd_hbm, w_hbm, out_hbm,
                  rows_v, sd_v, w_v, acc_sh, gsem):
    c = lax.axis_index("c")
    s = lax.axis_index("s")

    def _zero_row(r, _):
        for j in range(_VPR):
            rows_v[r, pl.ds(j * _LANES, _LANES)] = jnp.zeros(
                (_LANES,), jnp.float32)
        return 0

    lax.fori_loop(0, _ZR, _zero_row, 0)

    @pl.when(s < _NTEC - 1)
    def _zero_full():
        for r in range(_RPT // _ZR):
            pltpu.sync_copy(rows_v.at[pl.ds(0, _ZR)],
                            acc_sh.at[pl.ds(s * _RPT + r * _ZR, _ZR)])

    @pl.when(s == _NTEC - 1)
    def _zero_tail():
        for r in range((_N - (_NTEC - 1) * _RPT) // _ZR):
            pltpu.sync_copy(rows_v.at[pl.ds(0, _ZR)],
                            acc_sh.at[pl.ds(s * _RPT + r * _ZR, _ZR)])

    plsc.subcore_barrier()

    pltpu.sync_copy(w_hbm.at[c, s], w_v)

    def _slot(t, _):
        pltpu.sync_copy(sd_hbm.at[c, s, t], sd_v)
        pltpu.sync_copy(x_hbm.at[sd_v.at[0]], rows_v)

        def _group(g, _):
            wvec = w_v[pl.ds(t * _CHUNK + g * _LANES, _LANES)]
            for k in range(_LANES):
                e = g * _LANES + k
                wv = jnp.full((_LANES,), wvec[k])
                for j in range(_VPR):
                    sl = pl.ds(j * _LANES, _LANES)
                    rows_v[e, sl] = rows_v[e, sl] * wv
            return 0

        lax.fori_loop(0, _CHUNK // _LANES, _group, 0)
        pltpu.sync_copy(rows_v, acc_sh.at[sd_v.at[1]], add=True)
        return 0

    lax.fori_loop(0, _CPS, _slot, 0)
    plsc.subcore_barrier()

    @pl.when(s < _NTEC - 1)
    def _write_full():
        pltpu.sync_copy(acc_sh.at[pl.ds(s * _RPT, _RPT)],
                        out_hbm.at[c, pl.ds(s * _RPT, _RPT)])

    @pl.when(s == _NTEC - 1)
    def _write_tail():
        tail = _N - (_NTEC - 1) * _RPT
        pltpu.sync_copy(acc_sh.at[pl.ds((_NTEC - 1) * _RPT, tail)],
                        out_hbm.at[c, pl.ds((_NTEC - 1) * _RPT, tail)])


def _sc_spmm(x, sd, w):
    return pl.kernel(
        _sc_spmm_body,
        out_type=jax.ShapeDtypeStruct((_NSC, _N, _D), jnp.float32),
        mesh=plsc.VectorSubcoreMesh(core_axis_name="c",
                                    subcore_axis_name="s"),
        scratch_types=[
            pltpu.VMEM((_CHUNK, _D), jnp.float32),
            pltpu.VMEM((2, _CHUNK), jnp.int32),
            pltpu.VMEM((_CPS * _CHUNK,), jnp.float32),
            pltpu.VMEM_SHARED((_N, _D), jnp.float32),
            pltpu.SemaphoreType.DMA,
        ],
    )(x, sd, w)


_BLK = 1000
_SCALE = float(_D) ** 0.5


def _tc_readout_body(h_ref, a_ref, b_ref, w1_ref, b1_ref, w2_ref, b2_ref,
                     cw0_ref, cw1_ref, cb_ref, o_ref):
    hb = h_ref[...]
    a = a_ref[...]
    b = b_ref[...]
    t1 = jnp.maximum(
        jnp.dot(hb * a, w1_ref[...], preferred_element_type=jnp.float32)
        + b1_ref[...], 0.0)
    t2 = jnp.maximum(
        jnp.dot(hb * b, w2_ref[...], preferred_element_type=jnp.float32)
        + b2_ref[...], 0.0)
    s1 = jnp.sum(t1, axis=1, keepdims=True) * (1.0 / _SCALE)
    s2 = jnp.sum(t2, axis=1, keepdims=True) * (1.0 / _SCALE)
    m = jnp.maximum(s1, s2)
    e1 = jnp.exp(s1 - m)
    e2 = jnp.exp(s2 - m)
    r1 = e1 / (e1 + e2)
    nb = a * r1 + b * (1.0 - r1)
    o_ref[...] = hb * cw0_ref[...] + nb * cw1_ref[...] + cb_ref[...]


def _tc_readout(h, a, b, w1, b1, w2, b2, cw0, cw1, cb):
    full = pl.BlockSpec((_D, _D), lambda i: (0, 0))
    row = pl.BlockSpec((1, _D), lambda i: (0, 0))
    blk = pl.BlockSpec((_BLK, _D), lambda i: (i, 0))
    return pl.pallas_call(
        _tc_readout_body,
        grid=(_N // _BLK,),
        in_specs=[blk, blk, blk, full, row, full, row, row, row, row],
        out_specs=blk,
        out_shape=jax.ShapeDtypeStruct((_N, _D), jnp.float32),
    )(h, a, b, w1, b1, w2, b2, cw0, cw1, cb)


def kernel(x, edge_index_in, edge_weight_in, edge_index_out, edge_weight_out,
           W1_w, W1_b, W2_w, W2_b, conv_w, conv_b):
    pad = _CPS * _CHUNK * _NTEC - _E
    src = jnp.pad(jnp.stack([edge_index_in[1], edge_index_out[1]]),
                  ((0, 0), (0, pad))).reshape(_NSC, _NTEC, _CPS, 1, _CHUNK)
    dst = jnp.pad(jnp.stack([edge_index_in[0], edge_index_out[0]]),
                  ((0, 0), (0, pad))).reshape(_NSC, _NTEC, _CPS, 1, _CHUNK)
    sd = jnp.concatenate([src, dst], axis=3)
    w = jnp.pad(jnp.stack([edge_weight_in, edge_weight_out]),
                ((0, 0), (0, pad))).reshape(_NSC, _NTEC, _CPS * _CHUNK)

    nbrs = _sc_spmm(x, sd, w)

    out = _tc_readout(
        x, nbrs[0], nbrs[1],
        W1_w, W1_b.reshape(1, _D), W2_w, W2_b.reshape(1, _D),
        jnp.full((1, _D), conv_w[0]),
        jnp.full((1, _D), conv_w[1]),
        jnp.full((1, _D), conv_b))
    return out
